# baseline (device time: 15315 ns/iter reference)
import jax
import jax.numpy as jnp
from jax import lax
from jax.experimental import pallas as pl
from jax.experimental.pallas import tpu as pltpu

N_DEV = 4
R = 2


def kernel(A, B):
    m, k_per = A.shape
    _, n = B.shape
    blk = n // N_DEV
    mh = m // R

    def body(a_ref, b_ref, out_ref, b_bf,
             sb1, sb2, sbd, r1, r2, rd,
             agb, agr1, agr2, agrd,
             send_sems, recv_sems):
        me = lax.axis_index("i")
        p1 = me ^ 1
        p2 = 3 - me
        dg = me ^ 2

        barrier_sem = pltpu.get_barrier_semaphore()
        for nbr in [p1, p2, dg]:
            pl.semaphore_signal(
                barrier_sem, inc=1,
                device_id=(nbr,), device_id_type=pl.DeviceIdType.MESH,
            )

        a = a_ref[:, :].astype(jnp.bfloat16)
        b_bf[:, :] = b_ref[:, :].astype(jnp.bfloat16)

        def mm(col_off):
            return jnp.dot(a, b_bf[:, pl.ds(col_off, blk)],
                           preferred_element_type=jnp.float32)

        def xchg(sem_idx, src, dst, partner):
            return pltpu.make_async_remote_copy(
                src_ref=src, dst_ref=dst,
                send_sem=send_sems.at[sem_idx],
                recv_sem=recv_sems.at[sem_idx],
                device_id=(partner,),
                device_id_type=pl.DeviceIdType.MESH,
            )

        first = True
        rs_ops = {}
        for rel, (buf, rbuf, partner) in enumerate(
            ((sbd, rd, dg), (sb1, r1, p1), (sb2, r2, p2))
        ):
            buf[:, :] = mm(partner * blk).astype(jnp.bfloat16)
            if first:
                pl.semaphore_wait(barrier_sem, 3)
                first = False
            for h in range(R):
                rows = pl.ds(h * mh, mh)
                op = xchg(R * rel + h, buf.at[rows], rbuf.at[rows], partner)
                op.start()
                rs_ops[(rel, h)] = op

        own = mm(me * blk)

        ag_ops = {}
        for h in range(R):
            rows = pl.ds(h * mh, mh)
            rs_ops[(0, h)].wait()
            acc = (own[h * mh:(h + 1) * mh, :]
                   + rd[rows, :].astype(jnp.float32))
            rs_ops[(1, h)].wait()
            acc = acc + r1[rows, :].astype(jnp.float32)
            rs_ops[(2, h)].wait()
            mine = jnp.maximum(acc + r2[rows, :].astype(jnp.float32), 0.0)
            agb[rows, :] = mine.astype(jnp.bfloat16)
            for rel, (rbuf, partner) in enumerate(
                ((agrd, dg), (agr1, p1), (agr2, p2))
            ):
                op = xchg(3 * R + R * rel + h,
                          agb.at[rows], rbuf.at[rows], partner)
                op.start()
                ag_ops[(rel, h)] = op
            out_ref[rows, pl.ds(me * blk, blk)] = mine

        for h in range(R):
            rows = pl.ds(h * mh, mh)
            for rel, (rbuf, partner) in enumerate(
                ((agrd, dg), (agr1, p1), (agr2, p2))
            ):
                ag_ops[(rel, h)].wait()
                out_ref[rows, pl.ds(partner * blk, blk)] = (
                    rbuf[rows, :].astype(jnp.float32))

    comm = lambda: pltpu.VMEM((m, blk), jnp.bfloat16)
    return pl.pallas_call(
        body,
        out_shape=jax.ShapeDtypeStruct((m, n), jnp.float32),
        in_specs=[
            pl.BlockSpec(memory_space=pltpu.VMEM),
            pl.BlockSpec(memory_space=pltpu.VMEM),
        ],
        out_specs=pl.BlockSpec(memory_space=pltpu.VMEM),
        scratch_shapes=[
            pltpu.VMEM((k_per, n), jnp.bfloat16),
            comm(), comm(), comm(),
            comm(), comm(), comm(),
            comm(),
            comm(), comm(), comm(),
            pltpu.SemaphoreType.DMA((6 * R,)),
            pltpu.SemaphoreType.DMA((6 * R,)),
        ],
        compiler_params=pltpu.CompilerParams(collective_id=0),
    )(A, B)
